# baseline (speedup 1.0000x reference)
import functools

import jax
import jax.numpy as jnp
from jax import lax
from jax.experimental import pallas as pl
from jax.experimental.pallas import tpu as pltpu
from jax.experimental.pallas import tpu_sc as plsc

_N = 16384
_F = 128
_D = 64
_NW = 32
_RPW = _N // _NW
_CHUNK = 128


def _sc_gather(emb_table, idx2d):
    mesh = plsc.VectorSubcoreMesh(core_axis_name="core", subcore_axis_name="subcore")

    @functools.partial(
        pl.kernel,
        out_type=jax.ShapeDtypeStruct((_N, _D), jnp.float32),
        mesh=mesh,
        scratch_types=[
            pltpu.VMEM((_RPW // _CHUNK, _CHUNK), jnp.int32),
            pltpu.VMEM((_RPW, _D), jnp.float32),
            pltpu.SemaphoreType.DMA,
        ],
        compiler_params=pltpu.CompilerParams(use_tc_tiling_on_sc=False),
    )
    def gather_kernel(table_hbm, idx_hbm, out_hbm, idx_v, rows_v, sem):
        wid = lax.axis_index("subcore") * 2 + lax.axis_index("core")
        base = wid * _RPW
        pltpu.sync_copy(idx_hbm.at[pl.ds(wid * (_RPW // _CHUNK), _RPW // _CHUNK)], idx_v)
        for k in range(_RPW // _CHUNK):
            pltpu.async_copy(
                table_hbm.at[idx_v.at[k]],
                rows_v.at[pl.ds(k * _CHUNK, _CHUNK)],
                sem,
            )
        pltpu.make_async_copy(table_hbm.at[pl.ds(0, _RPW)], rows_v, sem).wait()
        pltpu.sync_copy(rows_v, out_hbm.at[pl.ds(base, _RPW)])

    return gather_kernel(emb_table, idx2d)


def _tc_body(x_ref, w_ref, b_ref, g_ref, o_ref):
    o_ref[...] = (
        jnp.dot(x_ref[...], w_ref[...], preferred_element_type=jnp.float32)
        + b_ref[...]
        + g_ref[...]
    )


def _tc_matmul_add(x, W, b2d, g):
    bn = 2048
    return pl.pallas_call(
        _tc_body,
        grid=(_N // bn,),
        in_specs=[
            pl.BlockSpec((bn, _F), lambda i: (i, 0)),
            pl.BlockSpec((_F, _D), lambda i: (0, 0)),
            pl.BlockSpec((1, _D), lambda i: (0, 0)),
            pl.BlockSpec((bn, _D), lambda i: (i, 0)),
        ],
        out_specs=pl.BlockSpec((bn, _D), lambda i: (i, 0)),
        out_shape=jax.ShapeDtypeStruct((_N, _D), jnp.float32),
    )(x, W, b2d, g)


def kernel(float_feat, id_feat, W, b, emb_table):
    idx2d = id_feat.astype(jnp.int32).reshape(_N // _CHUNK, _CHUNK)
    g = _sc_gather(emb_table, idx2d)
    return _tc_matmul_add(float_feat, W, b.reshape(1, _D), g)


# speedup vs baseline: 1.7071x; 1.7071x over previous
import functools

import jax
import jax.numpy as jnp
from jax import lax
from jax.experimental import pallas as pl
from jax.experimental.pallas import tpu as pltpu
from jax.experimental.pallas import tpu_sc as plsc

_N = 16384
_F = 128
_D = 64
_NW = 32
_RPW = _N // _NW
_CHUNK = 128


def _sc_gather(emb_table, idx):
    mesh = plsc.VectorSubcoreMesh(core_axis_name="core", subcore_axis_name="subcore")

    @functools.partial(
        pl.kernel,
        out_type=jax.ShapeDtypeStruct((_N, _D), jnp.float32),
        mesh=mesh,
        scratch_types=[
            pltpu.VMEM((_RPW,), jnp.int32),
            pltpu.VMEM((_RPW, _D), jnp.float32),
            pltpu.SemaphoreType.DMA,
        ],
    )
    def gather_kernel(table_hbm, idx_hbm, out_hbm, idx_v, rows_v, sem):
        wid = lax.axis_index("subcore") * 2 + lax.axis_index("core")
        base = wid * _RPW
        pltpu.sync_copy(idx_hbm.at[pl.ds(base, _RPW)], idx_v)

        @pl.loop(0, _RPW, step=16)
        def _issue(j0):
            vec = idx_v[pl.ds(j0, 16)]
            for l in range(16):
                pltpu.async_copy(table_hbm.at[vec[l]], rows_v.at[j0 + l], sem)

        pltpu.make_async_copy(table_hbm.at[pl.ds(0, _RPW)], rows_v, sem).wait()
        pltpu.sync_copy(rows_v, out_hbm.at[pl.ds(base, _RPW)])

    return gather_kernel(emb_table, idx)


def _tc_body(x_ref, w_ref, b_ref, g_ref, o_ref):
    o_ref[...] = (
        jnp.dot(x_ref[...], w_ref[...], preferred_element_type=jnp.float32)
        + b_ref[...]
        + g_ref[...]
    )


def _tc_matmul_add(x, W, b2d, g):
    bn = 2048
    return pl.pallas_call(
        _tc_body,
        grid=(_N // bn,),
        in_specs=[
            pl.BlockSpec((bn, _F), lambda i: (i, 0)),
            pl.BlockSpec((_F, _D), lambda i: (0, 0)),
            pl.BlockSpec((1, _D), lambda i: (0, 0)),
            pl.BlockSpec((bn, _D), lambda i: (i, 0)),
        ],
        out_specs=pl.BlockSpec((bn, _D), lambda i: (i, 0)),
        out_shape=jax.ShapeDtypeStruct((_N, _D), jnp.float32),
    )(x, W, b2d, g)


def kernel(float_feat, id_feat, W, b, emb_table):
    idx = id_feat.astype(jnp.int32)
    g = _sc_gather(emb_table, idx)
    return _tc_matmul_add(float_feat, W, b.reshape(1, _D), g)


# speedup vs baseline: 1.7210x; 1.0081x over previous
import functools

import jax
import jax.numpy as jnp
from jax import lax
from jax.experimental import pallas as pl
from jax.experimental.pallas import tpu as pltpu
from jax.experimental.pallas import tpu_sc as plsc

_N = 16384
_F = 128
_D = 64
_V = 1000000
_NW = 32
_BW = 256
_NFULL = _V // _BW
_BASE = _NFULL // _NW
_EXTRA = _NFULL - _BASE * _NW
_RCAP = _N + 16 * (_BASE + 2)
_SENTINEL = 2**30


def _popcount(mask):
    return plsc.all_reduce_population_count(mask)[0]


def _sc_sweep_select(table_t, tail_t, idx):
    mesh = plsc.VectorSubcoreMesh(core_axis_name="core", subcore_axis_name="subcore")

    @functools.partial(
        pl.kernel,
        out_type=(
            jax.ShapeDtypeStruct((_NW * _RCAP, 128), jnp.float32),
            jax.ShapeDtypeStruct((_NW * _RCAP + 2048,), jnp.int32),
            jax.ShapeDtypeStruct((512,), jnp.int32),
        ),
        mesh=mesh,
        scratch_types=[
            pltpu.VMEM((_N,), jnp.int32),
            pltpu.VMEM((_N + 16,), jnp.int32),
            pltpu.VMEM((_N + 16,), jnp.int32),
            pltpu.VMEM((_N + 16,), jnp.int32),
            pltpu.VMEM((_N + 16,), jnp.int32),
            pltpu.VMEM((_D, _BW), jnp.float32),
            pltpu.VMEM((_D, _V - _NFULL * _BW), jnp.float32),
            pltpu.VMEM((128, 128), jnp.float32),
            pltpu.VMEM((128,), jnp.int32),
            pltpu.VMEM((16,), jnp.int32),
            pltpu.SemaphoreType.DMA,
            pltpu.SemaphoreType.DMA,
        ],
        compiler_params=pltpu.CompilerParams(needs_layout_passes=False),
    )
    def sweep_kernel(
        table_hbm, tail_hbm, idx_hbm, rows_hbm, nlist_hbm, counts_hbm,
        idx_v, pj_v, pn_v, mj_v, mn_v, blk_v, blk2_v, stg_v, stgn_v, cnt_v, sem, osem,
    ):
        wid = lax.axis_index("subcore") * 2 + lax.axis_index("core")
        base = wid * _RCAP
        blo = wid * _BASE + jnp.minimum(wid, _EXTRA)
        bhi = blo + _BASE + jnp.where(wid < _EXTRA, 1, 0)
        is_last = wid == _NW - 1
        lanes = lax.iota(jnp.int32, 16)

        pltpu.sync_copy(idx_hbm, idx_v)

        def part_body(i, cnt):
            v = idx_v[pl.ds(i * 16, 16)]
            bid = lax.shift_right_logical(v, 8)
            m = (bid >= blo) & (bid < bhi)
            m = jnp.where(is_last, m | (bid == _NFULL), m)
            plsc.store_compressed(pj_v.at[pl.ds(cnt, 16)], v, mask=m)
            plsc.store_compressed(pn_v.at[pl.ds(cnt, 16)], lanes + i * 16, mask=m)
            return cnt + _popcount(m)

        mycnt = lax.fori_loop(0, _N // 16, part_body, jnp.int32(0))

        def process_block(b, slot, buf, wait_src):
            def scan_body(i, q):
                vj = pj_v[pl.ds(i * 16, 16)]
                vn = pn_v[pl.ds(i * 16, 16)]
                valid = (lanes + i * 16) < mycnt
                m = (lax.shift_right_logical(vj, 8) == b) & valid
                plsc.store_compressed(mj_v.at[pl.ds(q, 16)], vj & 255, mask=m)
                plsc.store_compressed(mn_v.at[pl.ds(q, 16)], vn, mask=m)
                return q + _popcount(m)

            q = lax.fori_loop(0, (mycnt + 15) // 16, scan_body, jnp.int32(0))
            pltpu.make_async_copy(wait_src, buf, sem).wait()

            def sel_body(k, slot2):
                r = (slot2 % 8) * 16

                @pl.when(slot2 >= 8)
                def _():
                    pltpu.make_async_copy(
                        rows_hbm.at[pl.ds(0, 16)], stg_v.at[pl.ds(r, 16)], osem
                    ).wait()
                    pltpu.make_async_copy(
                        nlist_hbm.at[pl.ds(0, 16)], stgn_v.at[pl.ds(r, 16)], osem
                    ).wait()

                m = lanes < (q - k * 16)
                jloc = jnp.where(m, mj_v[pl.ds(k * 16, 16)], 0)
                nvec = mn_v[pl.ds(k * 16, 16)]
                rowidx = lanes + r
                for f in range(_D):
                    vals = plsc.load_gather(
                        buf, [jnp.full((16,), f, jnp.int32), jloc]
                    )
                    plsc.store_scatter(
                        stg_v, [rowidx, jnp.full((16,), f, jnp.int32)], vals
                    )
                stgn_v[pl.ds(r, 16)] = jnp.where(m, nvec, _SENTINEL)
                dst = base + slot2 * 16
                pltpu.async_copy(
                    stg_v.at[pl.ds(r, 16)], rows_hbm.at[pl.ds(dst, 16)], osem
                )
                pltpu.async_copy(
                    stgn_v.at[pl.ds(r, 16)], nlist_hbm.at[pl.ds(dst, 16)], osem
                )
                return slot2 + 1

            return lax.fori_loop(0, (q + 15) // 16, sel_body, slot)

        def sweep_body(b, slot):
            pltpu.async_copy(table_hbm.at[:, pl.ds(b * _BW, _BW)], blk_v, sem)
            return process_block(b, slot, blk_v, table_hbm.at[:, pl.ds(0, _BW)])

        slot = lax.fori_loop(blo, bhi, sweep_body, jnp.int32(0))

        def last_fn(slot2):
            pltpu.async_copy(tail_hbm, blk2_v, sem)
            return process_block(jnp.int32(_NFULL), slot2, blk2_v, tail_hbm)

        slot = lax.cond(is_last, last_fn, lambda s: s, slot)

        @pl.loop(0, 8)
        def _drain(r):
            @pl.when(r < jnp.minimum(slot, 8))
            def _():
                pltpu.make_async_copy(
                    rows_hbm.at[pl.ds(0, 16)], stg_v.at[pl.ds(r * 16, 16)], osem
                ).wait()
                pltpu.make_async_copy(
                    nlist_hbm.at[pl.ds(0, 16)], stgn_v.at[pl.ds(r * 16, 16)], osem
                ).wait()

        cnt_v[...] = jnp.zeros((16,), jnp.int32) + slot * 16
        pltpu.sync_copy(cnt_v, counts_hbm.at[pl.ds(wid * 16, 16)])

    return sweep_kernel(table_t, tail_t, idx)


def _sc_permute(rows, nlist, counts):
    mesh = plsc.VectorSubcoreMesh(core_axis_name="core", subcore_axis_name="subcore")
    cpw = _N // _NW

    @functools.partial(
        pl.kernel,
        out_type=jax.ShapeDtypeStruct((_N, 128), jnp.float32),
        mesh=mesh,
        scratch_types=[
            pltpu.VMEM((512,), jnp.int32),
            pltpu.VMEM((2048,), jnp.int32),
            pltpu.VMEM((cpw + 16,), jnp.int32),
            pltpu.VMEM((cpw // 128, 128), jnp.int32),
            pltpu.VMEM((cpw, 128), jnp.float32),
            pltpu.SemaphoreType.DMA,
        ],
        compiler_params=pltpu.CompilerParams(needs_layout_passes=False),
    )
    def permute_kernel(
        rows_hbm, nlist_hbm, counts_hbm, g_hbm, cnts_v, nbuf_v, sf_v, s2_v, arr_v, sem
    ):
        wid = lax.axis_index("subcore") * 2 + lax.axis_index("core")
        nlo = wid * cpw
        lanes = lax.iota(jnp.int32, 16)
        pltpu.sync_copy(counts_hbm, cnts_v)

        for t in range(_NW):
            cvec = cnts_v[pl.ds(t * 16, 16)]
            c_t = cvec[0]
            rbase = t * _RCAP

            def chunk_body(ci, _):
                cb = ci * 2048
                pltpu.sync_copy(
                    nlist_hbm.at[pl.ds(rbase + cb, 2048)], nbuf_v
                )

                def scan_body(i, __):
                    v = nbuf_v[pl.ds(i * 16, 16)]
                    pos = cb + i * 16 + lanes
                    m = (v >= nlo) & (v < nlo + cpw) & (pos < c_t)
                    tgt = jnp.where(m, v - nlo, cpw + lanes)
                    plsc.store_scatter(sf_v, [tgt], rbase + pos)
                    return __

                nv = jnp.minimum(c_t - cb, 2048)
                lax.fori_loop(0, (nv + 15) // 16, scan_body, jnp.int32(0))
                return _

            lax.fori_loop(0, (c_t + 2047) // 2048, chunk_body, jnp.int32(0))

        @pl.loop(0, cpw // 16)
        def _copy(i):
            s2_v[i // 8, pl.ds((i % 8) * 16, 16)] = sf_v[pl.ds(i * 16, 16)]

        for k in range(cpw // 128):
            pltpu.async_copy(
                rows_hbm.at[s2_v.at[k]], arr_v.at[pl.ds(k * 128, 128)], sem
            )
        pltpu.make_async_copy(
            rows_hbm.at[pl.ds(0, cpw)], arr_v, sem
        ).wait()
        pltpu.sync_copy(arr_v, g_hbm.at[pl.ds(nlo, cpw)])

    return permute_kernel(rows, nlist, counts)


def _tc_body(x_ref, w_ref, b_ref, g_ref, o_ref):
    o_ref[...] = (
        jnp.dot(x_ref[...], w_ref[...], preferred_element_type=jnp.float32)
        + b_ref[...]
        + g_ref[...][:, :_D]
    )


def _tc_matmul_add(x, w, b2d, g):
    bn = 2048
    return pl.pallas_call(
        _tc_body,
        grid=(_N // bn,),
        in_specs=[
            pl.BlockSpec((bn, _F), lambda i: (i, 0)),
            pl.BlockSpec((_F, _D), lambda i: (0, 0)),
            pl.BlockSpec((1, _D), lambda i: (0, 0)),
            pl.BlockSpec((bn, 128), lambda i: (i, 0)),
        ],
        out_specs=pl.BlockSpec((bn, _D), lambda i: (i, 0)),
        out_shape=jax.ShapeDtypeStruct((_N, _D), jnp.float32),
    )(x, w, b2d, g)


def kernel(float_feat, id_feat, W, b, emb_table):
    idx = id_feat.astype(jnp.int32)
    rows, nlist, counts = _sc_sweep_select(
        emb_table.T, emb_table[_NFULL * _BW :, :].T, idx
    )
    g = _sc_permute(rows, nlist, counts)
    return _tc_matmul_add(float_feat, W, b.reshape(1, _D), g)


# speedup vs baseline: 3.3490x; 1.9460x over previous
import functools

import jax
import jax.numpy as jnp
from jax import lax
from jax.experimental import pallas as pl
from jax.experimental.pallas import tpu as pltpu
from jax.experimental.pallas import tpu_sc as plsc

_N = 16384
_F = 128
_D = 64
_V = 1000000
_NW = 32
_BW = 256
_NFULL = _V // _BW
_BASE = _NFULL // _NW
_EXTRA = _NFULL - _BASE * _NW
_G = _N + 2048
_SENTINEL_ROW = _N


def _popcount(mask):
    return plsc.all_reduce_population_count(mask)[0]


def _sc_gather_scatter(table_t, tail_t, idx):
    mesh = plsc.VectorSubcoreMesh(core_axis_name="core", subcore_axis_name="subcore")

    @functools.partial(
        pl.kernel,
        out_type=jax.ShapeDtypeStruct((_G, 128), jnp.float32),
        mesh=mesh,
        scratch_types=[
            pltpu.VMEM((_N + 16,), jnp.int32),
            pltpu.VMEM((_N + 16,), jnp.int32),
            pltpu.VMEM((_N + 16,), jnp.int32),
            pltpu.VMEM((_N + 16,), jnp.int32),
            pltpu.VMEM((_D, _BW), jnp.float32),
            pltpu.VMEM((_D, _BW), jnp.float32),
            pltpu.VMEM((_D, _V - _NFULL * _BW), jnp.float32),
            pltpu.VMEM((128, 128), jnp.float32),
            pltpu.VMEM((8, 16), jnp.int32),
            pltpu.SemaphoreType.DMA,
            pltpu.SemaphoreType.DMA,
            pltpu.SemaphoreType.DMA,
        ],
        compiler_params=pltpu.CompilerParams(needs_layout_passes=False),
    )
    def sweep_kernel(
        table_hbm, tail_hbm, idx_hbm, out_hbm,
        pj_v, pn_v, mj_v, mn_v, bufa_v, bufb_v, buft_v, stg_v, stgn_v,
        sema, semb, osem,
    ):
        wid = lax.axis_index("subcore") * 2 + lax.axis_index("core")
        blo = wid * _BASE + jnp.minimum(wid, _EXTRA)
        nb = _BASE + jnp.where(wid < _EXTRA, 1, 0)
        bhi = blo + nb
        is_last = wid == _NW - 1
        lanes = lax.iota(jnp.int32, 16)

        pltpu.sync_copy(idx_hbm, pj_v.at[pl.ds(0, _N)])

        def part_body(i, cnt):
            v = pj_v[pl.ds(i * 16, 16)]
            bid = lax.shift_right_logical(v, 8)
            m = (bid >= blo) & (bid < bhi)
            m = jnp.where(is_last, m | (bid == _NFULL), m)
            plsc.store_compressed(pj_v.at[pl.ds(cnt, 16)], v, mask=m)
            plsc.store_compressed(pn_v.at[pl.ds(cnt, 16)], lanes + i * 16, mask=m)
            return cnt + _popcount(m)

        mycnt = lax.fori_loop(0, _N // 16, part_body, jnp.int32(0))

        def process_block(b, slot, buf, wait_src, sem):
            def scan_body(i, q):
                vj = pj_v[pl.ds(i * 16, 16)]
                vn = pn_v[pl.ds(i * 16, 16)]
                valid = (lanes + i * 16) < mycnt
                m = (lax.shift_right_logical(vj, 8) == b) & valid
                plsc.store_compressed(mj_v.at[pl.ds(q, 16)], vj & 255, mask=m)
                plsc.store_compressed(mn_v.at[pl.ds(q, 16)], vn, mask=m)
                return q + _popcount(m)

            q = lax.fori_loop(0, (mycnt + 15) // 16, scan_body, jnp.int32(0))
            pltpu.make_async_copy(wait_src, buf, sem).wait()

            def sel_body(k, slot2):
                rr = slot2 % 8
                r = rr * 16

                @pl.when(slot2 >= 8)
                def _():
                    pltpu.make_async_copy(
                        out_hbm.at[pl.ds(0, 16)], stg_v.at[pl.ds(r, 16)], osem
                    ).wait()

                m = lanes < (q - k * 16)
                jloc = jnp.where(m, mj_v[pl.ds(k * 16, 16)], 0)
                nvec = mn_v[pl.ds(k * 16, 16)]
                rowidx = lanes + r
                for f in range(_D):
                    vals = plsc.load_gather(
                        buf, [jnp.full((16,), f, jnp.int32), jloc]
                    )
                    plsc.store_scatter(
                        stg_v, [rowidx, jnp.full((16,), f, jnp.int32)], vals
                    )
                pad_rows = _SENTINEL_ROW + (
                    (wid * 64 + slot2 * 16 + lanes) & (_G - _N - 1)
                )
                stgn_v[rr, :] = jnp.where(m, nvec, pad_rows)
                pltpu.async_copy(
                    stg_v.at[pl.ds(r, 16)], out_hbm.at[stgn_v.at[rr]], osem
                )
                return slot2 + 1

            return lax.fori_loop(0, (q + 15) // 16, sel_body, slot)

        full_src = table_hbm.at[:, pl.ds(0, _BW)]

        def issue(b, buf, sem):
            pltpu.async_copy(table_hbm.at[:, pl.ds(b * _BW, _BW)], buf, sem)

        issue(blo, bufa_v, sema)

        def pair_body(i, slot):
            b0 = blo + 2 * i
            b1 = b0 + 1

            @pl.when(b1 < bhi)
            def _():
                issue(b1, bufb_v, semb)

            slot = process_block(b0, slot, bufa_v, full_src, sema)

            @pl.when(b0 + 2 < bhi)
            def _():
                issue(b0 + 2, bufa_v, sema)

            return lax.cond(
                b1 < bhi,
                lambda s: process_block(b1, s, bufb_v, full_src, semb),
                lambda s: s,
                slot,
            )

        slot = lax.fori_loop(0, (nb + 1) // 2, pair_body, jnp.int32(0))

        def last_fn(slot2):
            pltpu.async_copy(tail_hbm, buft_v, sema)
            return process_block(jnp.int32(_NFULL), slot2, buft_v, tail_hbm, sema)

        slot = lax.cond(is_last, last_fn, lambda s: s, slot)

        @pl.loop(0, 8)
        def _drain(r):
            @pl.when(r < jnp.minimum(slot, 8))
            def _():
                pltpu.make_async_copy(
                    out_hbm.at[pl.ds(0, 16)], stg_v.at[pl.ds(r * 16, 16)], osem
                ).wait()

    return sweep_kernel(table_t, tail_t, idx)


def _tc_body(x_ref, w_ref, b_ref, g_ref, o_ref):
    o_ref[...] = (
        jnp.dot(x_ref[...], w_ref[...], preferred_element_type=jnp.float32)
        + b_ref[...]
        + g_ref[...][:, :_D]
    )


def _tc_matmul_add(x, w, b2d, g):
    bn = 2048
    return pl.pallas_call(
        _tc_body,
        grid=(_N // bn,),
        in_specs=[
            pl.BlockSpec((bn, _F), lambda i: (i, 0)),
            pl.BlockSpec((_F, _D), lambda i: (0, 0)),
            pl.BlockSpec((1, _D), lambda i: (0, 0)),
            pl.BlockSpec((bn, 128), lambda i: (i, 0)),
        ],
        out_specs=pl.BlockSpec((bn, _D), lambda i: (i, 0)),
        out_shape=jax.ShapeDtypeStruct((_N, _D), jnp.float32),
    )(x, w, b2d, g)


def kernel(float_feat, id_feat, W, b, emb_table):
    idx = id_feat.astype(jnp.int32)
    g = _sc_gather_scatter(emb_table.T, emb_table[_NFULL * _BW :, :].T, idx)
    return _tc_matmul_add(float_feat, W, b.reshape(1, _D), g)
